# baseline (speedup 1.0000x reference)
import dataclasses
import functools

import jax
import jax.numpy as jnp
from jax import lax
from jax.experimental import pallas as pl
from jax.experimental.pallas import tpu as pltpu
from jax.experimental.pallas import tpu_sc as plsc

NC = 2
NS = 16
NW = NC * NS
L = 16

def _sc_params(tc_tiling=True):
    cp = pltpu.CompilerParams()
    fields = pltpu.CompilerParams.__dataclass_fields__
    if "needs_layout_passes" in fields:
        cp = dataclasses.replace(cp, needs_layout_passes=False)
    if not tc_tiling and "use_tc_tiling_on_sc" in fields:
        cp = dataclasses.replace(cp, use_tc_tiling_on_sc=False)
    return cp


@functools.cache
def _sc_mesh():
    return plsc.VectorSubcoreMesh(core_axis_name="c", subcore_axis_name="s",
                                  num_cores=NC, num_subcores=NS)


def _deg_body(n_pad, epw, dst_hbm, out_hbm, dstbuf, hist, ones16, zeros16):
    cid = lax.axis_index("c")
    sid = lax.axis_index("s")
    wid = cid * NS + sid

    pltpu.sync_copy(dst_hbm.at[wid], dstbuf)

    @pl.loop(0, n_pad // L)
    def _(i):
        hist[pl.ds(i * L, L)] = zeros16

    @pl.loop(0, epw // L)
    def _(i):
        idx = dstbuf[pl.ds(i * L, L)]
        plsc.addupdate_scatter(hist, [idx], ones16)

    pltpu.sync_copy(hist, out_hbm.at[wid])


def _sc_degree(dst_flat, n_pad):
    nw, epw = dst_flat.shape

    def body(dst_hbm, out_hbm, dstbuf, hist):
        ones16 = jnp.full((L,), 1.0, jnp.float32)
        zeros16 = jnp.zeros((L,), jnp.float32)
        _deg_body(n_pad, epw, dst_hbm, out_hbm, dstbuf, hist, ones16, zeros16)

    return pl.kernel(
        body,
        out_type=jax.ShapeDtypeStruct((NW, n_pad), jnp.float32),
        mesh=_sc_mesh(),
        compiler_params=_sc_params(),
        scratch_types=[
            pltpu.VMEM((epw,), jnp.int32),
            pltpu.VMEM((n_pad,), jnp.float32),
        ],
    )(dst_flat)


NBUF = 2


def _unpack_idx(pkbuf, ch, sstage, dstage, c):
    for j in range(c // L):
        v = pkbuf[ch, pl.ds(j * L, L)]
        sstage[0, pl.ds(j * L, L)] = jnp.bitwise_and(v, 0xFFFF)
        dstage[0, pl.ds(j * L, L)] = lax.shift_right_logical(v, 16)


def _gsa_body(n_pad, nch, c, rows_hbm, pk_hbm, zeros_hbm, out_hbm,
              pkbuf, ss0, ss1, ds0, ds1, rb0, rb1,
              g0, g1, s0, s1, acc):
    rbufs = [rb0, rb1]
    sstages = [ss0, ss1]
    dstages = [ds0, ds1]
    gsems = [g0, g1]
    ssems = [s0, s1]
    cid = lax.axis_index("c")
    sid = lax.axis_index("s")
    wid = cid * NS + sid
    rpt = n_pad // NS

    pltpu.sync_copy(pk_hbm.at[wid], pkbuf)

    pltpu.sync_copy(zeros_hbm.at[pl.ds(sid * rpt, rpt)],
                    acc.at[pl.ds(sid * rpt, rpt)])
    plsc.subcore_barrier()

    for b in range(NBUF):
        _unpack_idx(pkbuf, b, sstages[b], dstages[b], c)
        pltpu.async_copy(rows_hbm.at[sstages[b].at[0]], rbufs[b], gsems[b])

    @pl.loop(0, nch // NBUF)
    def _(i):
        base = i * NBUF
        cps = []
        for b in range(NBUF):
            pltpu.make_async_copy(rows_hbm.at[sstages[b].at[0]],
                                  rbufs[b], gsems[b]).wait()
            cps.append(pltpu.async_copy(rbufs[b], acc.at[dstages[b].at[0]],
                                        ssems[b], add=True))
        for b in range(NBUF):
            cps[b].wait()
            _unpack_idx(pkbuf, base + NBUF + b, sstages[b], dstages[b], c)
            pltpu.async_copy(rows_hbm.at[sstages[b].at[0]], rbufs[b], gsems[b])

    for b in range(NBUF):
        pltpu.make_async_copy(rows_hbm.at[sstages[b].at[0]],
                              rbufs[b], gsems[b]).wait()

    plsc.subcore_barrier()
    pltpu.sync_copy(acc.at[pl.ds(sid * rpt, rpt)],
                    out_hbm.at[cid].at[pl.ds(sid * rpt, rpt)])


def _sc_gather_scatter_add(rows, pk2d, zeros, n_pad, d):
    nw, nchp, c = pk2d.shape
    nch = nchp - NBUF

    body = functools.partial(_gsa_body, n_pad, nch, c)
    return pl.kernel(
        body,
        out_type=jax.ShapeDtypeStruct((NC, n_pad, d), jnp.float32),
        mesh=_sc_mesh(),
        compiler_params=_sc_params(),
        scratch_types=[
            pltpu.VMEM((nchp, c), jnp.int32),
        ] + [pltpu.VMEM((1, c), jnp.int32) for _ in range(2 * NBUF)]
          + [pltpu.VMEM((c, d), jnp.float32) for _ in range(NBUF)]
          + [pltpu.SemaphoreType.DMA for _ in range(2 * NBUF)]
          + [pltpu.VMEM_SHARED((n_pad, d), jnp.float32)],
    )(rows, pk2d, zeros)


def _prescale_body(dp_ref, x_ref, xs_ref, dinv_ref):
    deg = jnp.sum(dp_ref[...], axis=0)
    dinv = lax.rsqrt(jnp.maximum(deg, 1.0))[:, None]
    xs_ref[...] = x_ref[...] * dinv
    dinv_ref[...] = dinv


def _tc_prescale(deg_parts, x_pad):
    n_pad, d = x_pad.shape
    return pl.pallas_call(
        _prescale_body,
        out_shape=(jax.ShapeDtypeStruct((n_pad, d), jnp.float32),
                   jax.ShapeDtypeStruct((n_pad, 1), jnp.float32)),
    )(deg_parts, x_pad)


def _mid_body(acc_ref, dinv_ref, w_ref, b_ref, out_ref):
    a = (acc_ref[0] + acc_ref[1]) * dinv_ref[...]
    h = jnp.dot(a.astype(jnp.bfloat16), w_ref[...].astype(jnp.bfloat16),
                preferred_element_type=jnp.float32)
    h = jnp.maximum(h + b_ref[...], 0.0)
    out_ref[...] = h * dinv_ref[...]


def _tc_mid(acc, dinv, w1, b1):
    _, n_pad, d = acc.shape
    return pl.pallas_call(
        _mid_body,
        out_shape=jax.ShapeDtypeStruct((n_pad, d), jnp.float32),
    )(acc, dinv, w1, b1.reshape(1, d))


def _final_body(acc_ref, dinv_ref, w_ref, b_ref, u_ref, out_ref):
    a = (acc_ref[0] + acc_ref[1]) * dinv_ref[...]
    o = jnp.dot(a.astype(jnp.bfloat16), w_ref[...].astype(jnp.bfloat16),
                preferred_element_type=jnp.float32) + b_ref[...]
    out_ref[...] = (u_ref[...] <= o[None]).astype(jnp.float32)


def _tc_final(acc, dinv, w2, b2, u):
    t, n, d = u.shape
    blk_n = 2000
    grid = (n // blk_n,)
    return pl.pallas_call(
        _final_body,
        grid=grid,
        in_specs=[
            pl.BlockSpec((NC, blk_n, d), lambda i: (0, i, 0)),
            pl.BlockSpec((blk_n, 1), lambda i: (i, 0)),
            pl.BlockSpec((d, d), lambda i: (0, 0)),
            pl.BlockSpec((1, d), lambda i: (0, 0)),
            pl.BlockSpec((t, blk_n, d), lambda i: (0, i, 0)),
        ],
        out_specs=pl.BlockSpec((t, blk_n, d), lambda i: (0, i, 0)),
        out_shape=jax.ShapeDtypeStruct((t, n, d), jnp.float32),
    )(acc, dinv, w2, b2.reshape(1, d), u)


def kernel(x, edge_index, W1, b1, W2, b2):
    n, d = x.shape
    e = edge_index.shape[1]
    t_steps = 4
    chunk = 128

    n_pad = ((n + 1 + NS * 8 - 1) // (NS * 8)) * (NS * 8)
    nch = -(-e // (NW * chunk))
    nch = ((nch + NBUF - 1) // NBUF) * NBUF
    epw = nch * chunk
    e_pad = epw * NW

    src = edge_index[0]
    dst = edge_index[1]
    src_p = jnp.concatenate([src, jnp.zeros((e_pad - e,), jnp.int32)])
    dst_p = jnp.concatenate([dst, jnp.full((e_pad - e,), n, jnp.int32)])
    packed = jnp.bitwise_or(src_p, jnp.left_shift(dst_p, 16))
    pk2d = jnp.concatenate(
        [packed.reshape(NW, nch, chunk),
         jnp.zeros((NW, NBUF, chunk), jnp.int32)], axis=1)
    dst_flat = dst_p.reshape(NW, epw)

    x_pad = jnp.pad(x, ((0, n_pad - n), (0, 0)))
    zeros = jnp.zeros((n_pad, d), jnp.float32)

    deg_parts = _sc_degree(dst_flat, n_pad)
    xs, dinv = _tc_prescale(deg_parts, x_pad)

    acc1 = _sc_gather_scatter_add(xs, pk2d, zeros, n_pad, d)
    hs = _tc_mid(acc1, dinv, W1, b1)

    acc2 = _sc_gather_scatter_add(hs, pk2d, zeros, n_pad, d)

    ekey = jax.random.key(42)
    u = jnp.stack([
        jax.random.uniform(jax.random.fold_in(ekey, t), (n, d),
                           dtype=jnp.float32)
        for t in range(t_steps)
    ])
    return _tc_final(acc2[:, :n], dinv[:n], W2, b2, u)


# speedup vs baseline: 1.0229x; 1.0229x over previous
import dataclasses
import functools

import jax
import jax.numpy as jnp
from jax import lax
from jax.experimental import pallas as pl
from jax.experimental.pallas import tpu as pltpu
from jax.experimental.pallas import tpu_sc as plsc

NC = 2
NS = 16
NW = NC * NS
L = 16

def _sc_params(tc_tiling=True):
    cp = pltpu.CompilerParams()
    fields = pltpu.CompilerParams.__dataclass_fields__
    if "needs_layout_passes" in fields:
        cp = dataclasses.replace(cp, needs_layout_passes=False)
    if not tc_tiling and "use_tc_tiling_on_sc" in fields:
        cp = dataclasses.replace(cp, use_tc_tiling_on_sc=False)
    return cp


@functools.cache
def _sc_mesh():
    return plsc.VectorSubcoreMesh(core_axis_name="c", subcore_axis_name="s",
                                  num_cores=NC, num_subcores=NS)


def _deg_body(n_pad, epw, dst_hbm, out_hbm, dstbuf, hist, ones16, zeros16):
    cid = lax.axis_index("c")
    sid = lax.axis_index("s")
    wid = cid * NS + sid

    pltpu.sync_copy(dst_hbm.at[wid], dstbuf)

    @pl.loop(0, n_pad // L)
    def _(i):
        hist[pl.ds(i * L, L)] = zeros16

    @pl.loop(0, epw // L)
    def _(i):
        idx = dstbuf[pl.ds(i * L, L)]
        plsc.addupdate_scatter(hist, [idx], ones16)

    pltpu.sync_copy(hist, out_hbm.at[wid])


def _sc_degree(dst_flat, n_pad):
    nw, epw = dst_flat.shape

    def body(dst_hbm, out_hbm, dstbuf, hist):
        ones16 = jnp.full((L,), 1.0, jnp.float32)
        zeros16 = jnp.zeros((L,), jnp.float32)
        _deg_body(n_pad, epw, dst_hbm, out_hbm, dstbuf, hist, ones16, zeros16)

    return pl.kernel(
        body,
        out_type=jax.ShapeDtypeStruct((NW, n_pad), jnp.float32),
        mesh=_sc_mesh(),
        compiler_params=_sc_params(),
        scratch_types=[
            pltpu.VMEM((epw,), jnp.int32),
            pltpu.VMEM((n_pad,), jnp.float32),
        ],
    )(dst_flat)


NBUF = 2


def _unpack_idx(pkbuf, ch, sstage, dstage, c):
    for j in range(c // L):
        v = pkbuf[ch, pl.ds(j * L, L)]
        sstage[0, pl.ds(j * L, L)] = jnp.bitwise_and(v, 0xFFFF)
        dstage[0, pl.ds(j * L, L)] = lax.shift_right_logical(v, 16)


def _gsa_body(n_pad, nch, c, rows_hbm, pk_hbm, zeros_hbm, out_hbm,
              pkbuf, ss0, ss1, ds0, ds1, rb0, rb1,
              g0, g1, s0, s1, acc):
    rbufs = [rb0, rb1]
    sstages = [ss0, ss1]
    dstages = [ds0, ds1]
    gsems = [g0, g1]
    ssems = [s0, s1]
    cid = lax.axis_index("c")
    sid = lax.axis_index("s")
    wid = cid * NS + sid
    rpt = n_pad // NS

    pltpu.sync_copy(pk_hbm.at[wid], pkbuf)

    pltpu.sync_copy(zeros_hbm.at[pl.ds(sid * rpt, rpt)],
                    acc.at[pl.ds(sid * rpt, rpt)])
    plsc.subcore_barrier()

    for b in range(NBUF):
        _unpack_idx(pkbuf, b, sstages[b], dstages[b], c)
        pltpu.async_copy(rows_hbm.at[sstages[b].at[0]], rbufs[b], gsems[b])

    @pl.loop(0, nch // NBUF)
    def _(i):
        base = i * NBUF
        for b in range(NBUF):
            pltpu.make_async_copy(rows_hbm.at[sstages[b].at[0]],
                                  rbufs[b], gsems[b]).wait()
            pltpu.sync_copy(rbufs[b], acc.at[dstages[b].at[0]], add=True)
            _unpack_idx(pkbuf, base + NBUF + b, sstages[b], dstages[b], c)
            pltpu.async_copy(rows_hbm.at[sstages[b].at[0]], rbufs[b], gsems[b])

    for b in range(NBUF):
        pltpu.make_async_copy(rows_hbm.at[sstages[b].at[0]],
                              rbufs[b], gsems[b]).wait()

    plsc.subcore_barrier()
    pltpu.sync_copy(acc.at[pl.ds(sid * rpt, rpt)],
                    out_hbm.at[cid].at[pl.ds(sid * rpt, rpt)])


def _sc_gather_scatter_add(rows, pk2d, zeros, n_pad, d):
    nw, nchp, c = pk2d.shape
    nch = nchp - NBUF

    body = functools.partial(_gsa_body, n_pad, nch, c)
    return pl.kernel(
        body,
        out_type=jax.ShapeDtypeStruct((NC, n_pad, d), jnp.float32),
        mesh=_sc_mesh(),
        compiler_params=_sc_params(),
        scratch_types=[
            pltpu.VMEM((nchp, c), jnp.int32),
        ] + [pltpu.VMEM((1, c), jnp.int32) for _ in range(2 * NBUF)]
          + [pltpu.VMEM((c, d), jnp.float32) for _ in range(NBUF)]
          + [pltpu.SemaphoreType.DMA for _ in range(2 * NBUF)]
          + [pltpu.VMEM_SHARED((n_pad, d), jnp.float32)],
    )(rows, pk2d, zeros)


def _prescale_body(dp_ref, x_ref, xs_ref, dinv_ref):
    deg = jnp.sum(dp_ref[...], axis=0)
    dinv = lax.rsqrt(jnp.maximum(deg, 1.0))[:, None]
    xs_ref[...] = x_ref[...] * dinv
    dinv_ref[...] = dinv


def _tc_prescale(deg_parts, x_pad):
    n_pad, d = x_pad.shape
    return pl.pallas_call(
        _prescale_body,
        out_shape=(jax.ShapeDtypeStruct((n_pad, d), jnp.float32),
                   jax.ShapeDtypeStruct((n_pad, 1), jnp.float32)),
    )(deg_parts, x_pad)


def _mid_body(acc_ref, dinv_ref, w_ref, b_ref, out_ref):
    a = (acc_ref[0] + acc_ref[1]) * dinv_ref[...]
    h = jnp.dot(a.astype(jnp.bfloat16), w_ref[...].astype(jnp.bfloat16),
                preferred_element_type=jnp.float32)
    h = jnp.maximum(h + b_ref[...], 0.0)
    out_ref[...] = h * dinv_ref[...]


def _tc_mid(acc, dinv, w1, b1):
    _, n_pad, d = acc.shape
    return pl.pallas_call(
        _mid_body,
        out_shape=jax.ShapeDtypeStruct((n_pad, d), jnp.float32),
    )(acc, dinv, w1, b1.reshape(1, d))


def _final_body(acc_ref, dinv_ref, w_ref, b_ref, u_ref, out_ref):
    a = (acc_ref[0] + acc_ref[1]) * dinv_ref[...]
    o = jnp.dot(a.astype(jnp.bfloat16), w_ref[...].astype(jnp.bfloat16),
                preferred_element_type=jnp.float32) + b_ref[...]
    out_ref[...] = (u_ref[...] <= o[None]).astype(jnp.float32)


def _tc_final(acc, dinv, w2, b2, u):
    t, n, d = u.shape
    blk_n = 2000
    grid = (n // blk_n,)
    return pl.pallas_call(
        _final_body,
        grid=grid,
        in_specs=[
            pl.BlockSpec((NC, blk_n, d), lambda i: (0, i, 0)),
            pl.BlockSpec((blk_n, 1), lambda i: (i, 0)),
            pl.BlockSpec((d, d), lambda i: (0, 0)),
            pl.BlockSpec((1, d), lambda i: (0, 0)),
            pl.BlockSpec((t, blk_n, d), lambda i: (0, i, 0)),
        ],
        out_specs=pl.BlockSpec((t, blk_n, d), lambda i: (0, i, 0)),
        out_shape=jax.ShapeDtypeStruct((t, n, d), jnp.float32),
    )(acc, dinv, w2, b2.reshape(1, d), u)


def kernel(x, edge_index, W1, b1, W2, b2):
    n, d = x.shape
    e = edge_index.shape[1]
    t_steps = 4
    chunk = 128

    n_pad = ((n + 1 + NS * 8 - 1) // (NS * 8)) * (NS * 8)
    nch = -(-e // (NW * chunk))
    nch = ((nch + NBUF - 1) // NBUF) * NBUF
    epw = nch * chunk
    e_pad = epw * NW

    src = edge_index[0]
    dst = edge_index[1]
    src_p = jnp.concatenate([src, jnp.zeros((e_pad - e,), jnp.int32)])
    dst_p = jnp.concatenate([dst, jnp.full((e_pad - e,), n, jnp.int32)])
    packed = jnp.bitwise_or(src_p, jnp.left_shift(dst_p, 16))
    pk2d = jnp.concatenate(
        [packed.reshape(NW, nch, chunk),
         jnp.zeros((NW, NBUF, chunk), jnp.int32)], axis=1)
    dst_flat = dst_p.reshape(NW, epw)

    x_pad = jnp.pad(x, ((0, n_pad - n), (0, 0)))
    zeros = jnp.zeros((n_pad, d), jnp.float32)

    deg_parts = _sc_degree(dst_flat, n_pad)
    xs, dinv = _tc_prescale(deg_parts, x_pad)

    acc1 = _sc_gather_scatter_add(xs, pk2d, zeros, n_pad, d)
    hs = _tc_mid(acc1, dinv, W1, b1)

    acc2 = _sc_gather_scatter_add(hs, pk2d, zeros, n_pad, d)

    ekey = jax.random.key(42)
    u = jnp.stack([
        jax.random.uniform(jax.random.fold_in(ekey, t), (n, d),
                           dtype=jnp.float32)
        for t in range(t_steps)
    ])
    return _tc_final(acc2[:, :n], dinv[:n], W2, b2, u)


# speedup vs baseline: 1.9889x; 1.9443x over previous
import dataclasses
import functools

import jax
import jax.numpy as jnp
from jax import lax
from jax.experimental import pallas as pl
from jax.experimental.pallas import tpu as pltpu
from jax.experimental.pallas import tpu_sc as plsc

NC = 2
NS = 16
NW = NC * NS
L = 16

def _sc_params(tc_tiling=True):
    cp = pltpu.CompilerParams()
    fields = pltpu.CompilerParams.__dataclass_fields__
    if "needs_layout_passes" in fields:
        cp = dataclasses.replace(cp, needs_layout_passes=False)
    if not tc_tiling and "use_tc_tiling_on_sc" in fields:
        cp = dataclasses.replace(cp, use_tc_tiling_on_sc=False)
    return cp


@functools.cache
def _sc_mesh():
    return plsc.VectorSubcoreMesh(core_axis_name="c", subcore_axis_name="s",
                                  num_cores=NC, num_subcores=NS)


def _deg_body(n_pad, epw, dst_hbm, out_hbm, dstbuf, hist, ones16, zeros16):
    cid = lax.axis_index("c")
    sid = lax.axis_index("s")
    wid = cid * NS + sid

    pltpu.sync_copy(dst_hbm.at[wid], dstbuf)

    @pl.loop(0, n_pad // L)
    def _(i):
        hist[pl.ds(i * L, L)] = zeros16

    @pl.loop(0, epw // L)
    def _(i):
        idx = dstbuf[pl.ds(i * L, L)]
        plsc.addupdate_scatter(hist, [idx], ones16)

    pltpu.sync_copy(hist, out_hbm.at[wid])


def _sc_degree(dst_flat, n_pad):
    nw, epw = dst_flat.shape

    def body(dst_hbm, out_hbm, dstbuf, hist):
        ones16 = jnp.full((L,), 1.0, jnp.float32)
        zeros16 = jnp.zeros((L,), jnp.float32)
        _deg_body(n_pad, epw, dst_hbm, out_hbm, dstbuf, hist, ones16, zeros16)

    return pl.kernel(
        body,
        out_type=jax.ShapeDtypeStruct((NW, n_pad), jnp.float32),
        mesh=_sc_mesh(),
        compiler_params=_sc_params(),
        scratch_types=[
            pltpu.VMEM((epw,), jnp.int32),
            pltpu.VMEM((n_pad,), jnp.float32),
        ],
    )(dst_flat)


def _gsa_body(n_pad, nch, rows_hbm, src_hbm, dst_hbm, zeros_hbm, out_hbm,
              srcbuf, dstbuf, rows_v, acc):
    cid = lax.axis_index("c")
    sid = lax.axis_index("s")
    wid = cid * NS + sid
    rpt = n_pad // NS

    pltpu.sync_copy(src_hbm.at[wid], srcbuf)
    pltpu.sync_copy(dst_hbm.at[wid], dstbuf)

    pltpu.sync_copy(zeros_hbm.at[pl.ds(sid * rpt, rpt)],
                    acc.at[pl.ds(sid * rpt, rpt)])
    plsc.subcore_barrier()

    @pl.loop(0, nch)
    def _(ch):
        pltpu.sync_copy(rows_hbm.at[srcbuf.at[ch]], rows_v)
        pltpu.sync_copy(rows_v, acc.at[dstbuf.at[ch]], add=True)

    plsc.subcore_barrier()
    pltpu.sync_copy(acc.at[pl.ds(sid * rpt, rpt)],
                    out_hbm.at[cid].at[pl.ds(sid * rpt, rpt)])


def _sc_gather_scatter_add(rows, src2d, dst2d, zeros, n_pad, d):
    nw, nch, c = src2d.shape

    body = functools.partial(_gsa_body, n_pad, nch)
    return pl.kernel(
        body,
        out_type=jax.ShapeDtypeStruct((NC, n_pad, d), jnp.float32),
        mesh=_sc_mesh(),
        compiler_params=_sc_params(),
        scratch_types=[
            pltpu.VMEM((nch, c), jnp.int32),
            pltpu.VMEM((nch, c), jnp.int32),
            pltpu.VMEM((c, d), jnp.float32),
            pltpu.VMEM_SHARED((n_pad, d), jnp.float32),
        ],
    )(rows, src2d, dst2d, zeros)


def _prescale_body(dp_ref, x_ref, xs_ref, dinv_ref):
    deg = jnp.sum(dp_ref[...], axis=0)
    dinv = lax.rsqrt(jnp.maximum(deg, 1.0))[:, None]
    xs_ref[...] = x_ref[...] * dinv
    dinv_ref[...] = dinv


def _tc_prescale(deg_parts, x_pad):
    n_pad, d = x_pad.shape
    return pl.pallas_call(
        _prescale_body,
        out_shape=(jax.ShapeDtypeStruct((n_pad, d), jnp.float32),
                   jax.ShapeDtypeStruct((n_pad, 1), jnp.float32)),
    )(deg_parts, x_pad)


def _mid_body(acc_ref, dinv_ref, w_ref, b_ref, out_ref):
    a = (acc_ref[0] + acc_ref[1]) * dinv_ref[...]
    h = jnp.dot(a.astype(jnp.bfloat16), w_ref[...].astype(jnp.bfloat16),
                preferred_element_type=jnp.float32)
    h = jnp.maximum(h + b_ref[...], 0.0)
    out_ref[...] = h * dinv_ref[...]


def _tc_mid(acc, dinv, w1, b1):
    _, n_pad, d = acc.shape
    return pl.pallas_call(
        _mid_body,
        out_shape=jax.ShapeDtypeStruct((n_pad, d), jnp.float32),
    )(acc, dinv, w1, b1.reshape(1, d))


def _final_body(acc_ref, dinv_ref, w_ref, b_ref, u_ref, out_ref):
    a = (acc_ref[0] + acc_ref[1]) * dinv_ref[...]
    o = jnp.dot(a.astype(jnp.bfloat16), w_ref[...].astype(jnp.bfloat16),
                preferred_element_type=jnp.float32) + b_ref[...]
    out_ref[...] = (u_ref[...] <= o[None]).astype(jnp.float32)


def _tc_final(acc, dinv, w2, b2, u):
    t, n, d = u.shape
    blk_n = 2000
    grid = (n // blk_n,)
    return pl.pallas_call(
        _final_body,
        grid=grid,
        in_specs=[
            pl.BlockSpec((NC, blk_n, d), lambda i: (0, i, 0)),
            pl.BlockSpec((blk_n, 1), lambda i: (i, 0)),
            pl.BlockSpec((d, d), lambda i: (0, 0)),
            pl.BlockSpec((1, d), lambda i: (0, 0)),
            pl.BlockSpec((t, blk_n, d), lambda i: (0, i, 0)),
        ],
        out_specs=pl.BlockSpec((t, blk_n, d), lambda i: (0, i, 0)),
        out_shape=jax.ShapeDtypeStruct((t, n, d), jnp.float32),
    )(acc, dinv, w2, b2.reshape(1, d), u)


def kernel(x, edge_index, W1, b1, W2, b2):
    n, d = x.shape
    e = edge_index.shape[1]
    t_steps = 4
    chunk = 128

    n_pad = ((n + 1 + NS * 8 - 1) // (NS * 8)) * (NS * 8)
    nch = -(-e // (NW * chunk))
    epw = nch * chunk
    e_pad = epw * NW

    src = edge_index[0]
    dst = edge_index[1]
    src_p = jnp.concatenate([src, jnp.zeros((e_pad - e,), jnp.int32)])
    dst_p = jnp.concatenate([dst, jnp.full((e_pad - e,), n, jnp.int32)])
    src2d = src_p.reshape(NW, nch, chunk)
    dst2d = dst_p.reshape(NW, nch, chunk)
    dst_flat = dst_p.reshape(NW, epw)

    x_pad = jnp.pad(x, ((0, n_pad - n), (0, 0)))
    zeros = jnp.zeros((n_pad, d), jnp.float32)

    deg_parts = _sc_degree(dst_flat, n_pad)
    xs, dinv = _tc_prescale(deg_parts, x_pad)

    acc1 = _sc_gather_scatter_add(xs, src2d, dst2d, zeros, n_pad, d)
    hs = _tc_mid(acc1, dinv, W1, b1)

    acc2 = _sc_gather_scatter_add(hs, src2d, dst2d, zeros, n_pad, d)

    ekey = jax.random.key(42)
    u = jnp.stack([
        jax.random.uniform(jax.random.fold_in(ekey, t), (n, d),
                           dtype=jnp.float32)
        for t in range(t_steps)
    ])
    return _tc_final(acc2[:, :n], dinv[:n], W2, b2, u)
